# baseline (speedup 1.0000x reference)
import jax
import jax.numpy as jnp
from jax.experimental import pallas as pl
from jax.experimental.pallas import tpu as pltpu

_B, _S, _F, _D = 64, 129, 768, 4
_MOM = 0.9
_EPS = 1e-6
_TS = 16
_NC = 9
_NZB = 3
_XOB = 3
_R = 3 * _B
_BIG = 1e30


def _chunk_w(c):
    return _TS if c < _NC - 1 else 1


def _main_kernel(x_hbm, nz_hbm, mbuf_ref, vbuf_ref,
                 lm_ref, dom_ref, ds_ref, lnr_ref, lnc_ref,
                 xmix_hbm, nm_ref, nv_ref, loss_ref,
                 xs_ref, nzb_ref, xob_ref, coef_ref,
                 xsem, nzsem, osem):

    def x_copy(c):
        w = _chunk_w(c)
        return pltpu.make_async_copy(
            x_hbm.at[:, pl.ds(c * _TS, w), :],
            xs_ref.at[c, :, 0:w, :], xsem.at[c])

    def nz_copy(c):
        w = _chunk_w(c)
        return pltpu.make_async_copy(
            nz_hbm.at[:, pl.ds(c * _TS, w), :],
            nzb_ref.at[c % _NZB, :, 0:w, :], nzsem.at[c % _NZB])

    def out_copy(c):
        w = _chunk_w(c)
        return pltpu.make_async_copy(
            xob_ref.at[c % _XOB, :, 0:w, :],
            xmix_hbm.at[:, pl.ds(c * _TS, w), :], osem.at[c % _XOB])

    for c in range(_NC):
        x_copy(c).start()
    for c in range(_NZB):
        nz_copy(c).start()

    for c in range(_NC):
        x_copy(c).wait()
    sum1 = jnp.zeros((_B, _F), jnp.float32)
    sum2 = jnp.zeros((_B, _F), jnp.float32)
    for c in range(_NC):
        xc = xs_ref[c, :, 0:_chunk_w(c), :]
        sum1 = sum1 + jnp.sum(xc, axis=1)
        sum2 = sum2 + jnp.sum(xc * xc, axis=1)

    mean_buf = mbuf_ref[...]
    var_buf = vbuf_ref[...]
    domc = dom_ref[...]
    dsc = ds_ref[...]

    nm_rows = []
    nv_rows = []
    for d in range(_D):
        mask = jnp.where(domc == float(d), 1.0, 0.0)
        nb = jnp.sum(mask, axis=0, keepdims=True)
        s1d = jnp.sum(sum1 * mask, axis=0, keepdims=True)
        s2d = jnp.sum(sum2 * mask, axis=0, keepdims=True)
        cnt = nb * float(_S)
        mu = s1d / jnp.maximum(cnt, 1.0)
        var = (s2d - cnt * mu * mu) / jnp.maximum(cnt - 1.0, 1.0)
        present = nb > 0.0
        mb = mean_buf[d:d + 1, :]
        vb = var_buf[d:d + 1, :]
        nm_rows.append(jnp.where(present, _MOM * mb + (1.0 - _MOM) * mu, mb))
        nv_rows.append(jnp.where(present, _MOM * vb + (1.0 - _MOM) * var, vb))
    new_mean = jnp.concatenate(nm_rows, axis=0)
    new_var = jnp.concatenate(nv_rows, axis=0)
    nm_ref[...] = new_mean
    nv_ref[...] = new_var

    sig = jnp.sqrt(new_var + _EPS)
    mu_ds = jnp.zeros((_B, _F), jnp.float32)
    sg_ds = jnp.zeros((_B, _F), jnp.float32)
    mu_dm = jnp.zeros((_B, _F), jnp.float32)
    sg_dm = jnp.zeros((_B, _F), jnp.float32)
    for d in range(_D):
        m_row = jnp.broadcast_to(new_mean[d:d + 1, :], (_B, _F))
        s_row = jnp.broadcast_to(sig[d:d + 1, :], (_B, _F))
        sel_ds = dsc == float(d)
        sel_dm = domc == float(d)
        mu_ds = jnp.where(sel_ds, m_row, mu_ds)
        sg_ds = jnp.where(sel_ds, s_row, sg_ds)
        mu_dm = jnp.where(sel_dm, m_row, mu_dm)
        sg_dm = jnp.where(sel_dm, s_row, sg_dm)

    mu_i = sum1 * (1.0 / float(_S))
    v_i = (sum2 - float(_S) * mu_i * mu_i) * (1.0 / float(_S - 1))
    inv = jax.lax.rsqrt(v_i + _EPS)
    lm = lm_ref[...]
    a = sg_ds * inv
    coef_ref[0] = lm + (1.0 - lm) * a
    coef_ref[1] = (1.0 - lm) * (mu_ds - a * mu_i)
    coef_ref[2] = sg_dm
    coef_ref[3] = mu_dm

    acc0 = None
    acc1 = None
    for c in range(_NC):
        nz_copy(c).wait()
        if c >= _XOB:
            out_copy(c - _XOB).wait()
        for t in range(_chunk_w(c)):
            xt = xs_ref[c][:, t, :]
            mt = coef_ref[0] * xt + coef_ref[1]
            ht = coef_ref[2] * nzb_ref[c % _NZB][:, t, :] + coef_ref[3]
            xob_ref[c % _XOB, :, t, :] = mt
            rows = jnp.concatenate([xt, mt, ht], axis=0)
            p = jax.lax.dot_general(rows, rows, (((1,), (1,)), ((), ())),
                                    preferred_element_type=jnp.float32)
            if (c * _TS + t) % 2 == 0:
                acc0 = p if acc0 is None else acc0 + p
            else:
                acc1 = p if acc1 is None else acc1 + p
        out_copy(c).start()
        if c + _NZB < _NC:
            nz_copy(c + _NZB).start()
    g = acc0 + acc1

    ri = jax.lax.broadcasted_iota(jnp.int32, (_R, _R), 0)
    ci = jax.lax.broadcasted_iota(jnp.int32, (_R, _R), 1)
    gd = jnp.where(ri == ci, g, 0.0)
    sqc = jnp.sum(gd, axis=1, keepdims=True)
    sqr = jnp.sum(gd, axis=0, keepdims=True)
    d2 = sqc + sqr - 2.0 * g
    dist = jnp.sqrt(jnp.maximum(d2, 1e-12))
    pos = lnc_ref[...] == lnr_ref[...]
    ap = jnp.max(jnp.where(pos, dist, -_BIG), axis=1, keepdims=True)
    an = jnp.min(jnp.where(pos, _BIG, dist), axis=1, keepdims=True)
    z = ap - an
    sp = jnp.maximum(z, 0.0) + jnp.log(1.0 + jnp.exp(-jnp.abs(z)))
    loss_ref[...] = jnp.sum(sp, axis=0, keepdims=True) * (1.0 / float(_R))

    for c in range(max(_NC - _XOB, 0), _NC):
        out_copy(c).wait()


def kernel(input, lmda, mean_buf, var_buf, hg_noise, labels, domain, d_rand):
    x = input
    f32 = jnp.float32

    domf = domain.astype(f32).reshape(_B, 1)
    dsf = ((domain + d_rand) % _D).astype(f32).reshape(_B, 1)
    lmf = lmda.astype(f32).reshape(_B, 1)
    ln = jnp.concatenate([labels, labels, -jnp.ones((_B,), labels.dtype)])
    lnf = ln.astype(f32)
    lnr = lnf.reshape(1, _R)
    lnc = lnf.reshape(_R, 1)

    x_mix, new_mean, new_var, loss = pl.pallas_call(
        _main_kernel,
        in_specs=[
            pl.BlockSpec(memory_space=pl.ANY),
            pl.BlockSpec(memory_space=pl.ANY),
            pl.BlockSpec((_D, _F), lambda: (0, 0)),
            pl.BlockSpec((_D, _F), lambda: (0, 0)),
            pl.BlockSpec((_B, 1), lambda: (0, 0)),
            pl.BlockSpec((_B, 1), lambda: (0, 0)),
            pl.BlockSpec((_B, 1), lambda: (0, 0)),
            pl.BlockSpec((1, _R), lambda: (0, 0)),
            pl.BlockSpec((_R, 1), lambda: (0, 0)),
        ],
        out_specs=[
            pl.BlockSpec(memory_space=pl.ANY),
            pl.BlockSpec((_D, _F), lambda: (0, 0)),
            pl.BlockSpec((_D, _F), lambda: (0, 0)),
            pl.BlockSpec((1, 1), lambda: (0, 0)),
        ],
        out_shape=[
            jax.ShapeDtypeStruct((_B, _S, _F), f32),
            jax.ShapeDtypeStruct((_D, _F), f32),
            jax.ShapeDtypeStruct((_D, _F), f32),
            jax.ShapeDtypeStruct((1, 1), f32),
        ],
        scratch_shapes=[
            pltpu.VMEM((_NC, _B, _TS, _F), f32),
            pltpu.VMEM((_NZB, _B, _TS, _F), f32),
            pltpu.VMEM((_XOB, _B, _TS, _F), f32),
            pltpu.VMEM((4, _B, _F), f32),
            pltpu.SemaphoreType.DMA((_NC,)),
            pltpu.SemaphoreType.DMA((_NZB,)),
            pltpu.SemaphoreType.DMA((_XOB,)),
        ],
        compiler_params=pltpu.CompilerParams(
            vmem_limit_bytes=55 * 1024 * 1024),
        name="domainmix_fused",
    )(x, hg_noise, mean_buf, var_buf, lmf, domf, dsf, lnr, lnc)

    return x_mix, loss[0, 0], new_mean, new_var
